# baseline (speedup 1.0000x reference)
import functools

import jax
import jax.numpy as jnp
from jax import lax
from jax.experimental import pallas as pl
from jax.experimental.pallas import tpu as pltpu
from jax.experimental.pallas import tpu_sc as plsc

N = 10000
E = 320000
D = 128
H = 128

BN = 2000
BE = 2000

CH = 2
E2 = E // CH
NW = 32
EPW = E2 // NW
GC = 128
NFULL = EPW // GC
REM = EPW - NFULL * GC


def _silu(x):
    return x * jax.nn.sigmoid(x)


def _pre_body(h_ref, wa_ref, wb_ref, a_ref, b_ref):
    h = h_ref[...]
    a_ref[...] = jnp.dot(h, wa_ref[...],
                         preferred_element_type=jnp.float32).astype(jnp.bfloat16)
    b_ref[...] = jnp.dot(h, wb_ref[...],
                         preferred_element_type=jnp.float32).astype(jnp.bfloat16)


def _precompute_ab(h, WaT, WbT):
    grid = (N // BN,)
    return pl.pallas_call(
        _pre_body,
        grid=grid,
        in_specs=[
            pl.BlockSpec((BN, D), lambda i: (i, 0)),
            pl.BlockSpec((D, H), lambda i: (0, 0)),
            pl.BlockSpec((D, H), lambda i: (0, 0)),
        ],
        out_specs=[
            pl.BlockSpec((BN, H), lambda i: (i, 0)),
            pl.BlockSpec((BN, H), lambda i: (i, 0)),
        ],
        out_shape=[
            jax.ShapeDtypeStruct((N, H), jnp.bfloat16),
            jax.ShapeDtypeStruct((N, H), jnp.bfloat16),
        ],
    )(h, WaT, WbT)


def _sc_gather_body(a_t, b_t, c_t, rowi, coli, g1, g2, c1, c2,
                    idx1_l, idx2_l, g1_v, g2_v, c1_v, c2_v,
                    sem1, sem2, sem3, sem4, sem5, sem6, sem7, sem8,
                    sem9, sem10, sem11, sem12, sem13, sem14, sem15, sem16):
    wid = lax.axis_index("c") * 16 + lax.axis_index("s")
    base = wid * EPW

    pltpu.sync_copy(rowi.at[pl.ds(base, EPW)], idx1_l)
    pltpu.sync_copy(coli.at[pl.ds(base, EPW)], idx2_l)

    def gather_issue(i, slot, sems, n):
        off = i * GC
        i1 = idx1_l.at[pl.ds(off, n)]
        i2 = idx2_l.at[pl.ds(off, n)]
        return (
            pltpu.async_copy(a_t.at[i1], g1_v.at[slot].at[pl.ds(0, n)], sems[0]),
            pltpu.async_copy(b_t.at[i2], g2_v.at[slot].at[pl.ds(0, n)], sems[1]),
            pltpu.async_copy(c_t.at[i1], c1_v.at[slot].at[pl.ds(0, n)], sems[2]),
            pltpu.async_copy(c_t.at[i2], c2_v.at[slot].at[pl.ds(0, n)], sems[3]),
        )

    def wb_issue(i, slot, sems, n):
        off = base + i * GC
        return (
            pltpu.async_copy(g1_v.at[slot].at[pl.ds(0, n)], g1.at[pl.ds(off, n)], sems[0]),
            pltpu.async_copy(g2_v.at[slot].at[pl.ds(0, n)], g2.at[pl.ds(off, n)], sems[1]),
            pltpu.async_copy(c1_v.at[slot].at[pl.ds(0, n)], c1.at[pl.ds(off, n)], sems[2]),
            pltpu.async_copy(c2_v.at[slot].at[pl.ds(0, n)], c2.at[pl.ds(off, n)], sems[3]),
        )

    ga_sems = (sem1, sem2, sem3, sem4)
    gb_sems = (sem5, sem6, sem7, sem8)
    wa_sems = (sem9, sem10, sem11, sem12)
    wb_sems = (sem13, sem14, sem15, sem16)

    def pair(j, carry):
        i0 = 2 * j
        i1 = i0 + 1
        ga = gather_issue(i0, 0, ga_sems, GC)
        gb = gather_issue(i1, 1, gb_sems, GC)
        for cp in ga:
            cp.wait()
        wa = wb_issue(i0, 0, wa_sems, GC)
        for cp in gb:
            cp.wait()
        wb = wb_issue(i1, 1, wb_sems, GC)
        for cp in wa:
            cp.wait()
        for cp in wb:
            cp.wait()
        return carry

    lax.fori_loop(0, NFULL // 2, pair, 0)

    for i, n in ([(NFULL - 1, GC)] if NFULL % 2 else []) + [(NFULL, REM)]:
        ga = gather_issue(i, 0, ga_sems, n)
        for cp in ga:
            cp.wait()
        wa = wb_issue(i, 0, wa_sems, n)
        for cp in wa:
            cp.wait()


@functools.cache
def _sc_gather():
    return functools.partial(
        pl.kernel,
        out_type=[jax.ShapeDtypeStruct((E2, H), jnp.bfloat16),
                  jax.ShapeDtypeStruct((E2, H), jnp.bfloat16),
                  jax.ShapeDtypeStruct((E2, 16), jnp.float32),
                  jax.ShapeDtypeStruct((E2, 16), jnp.float32)],
        mesh=plsc.VectorSubcoreMesh(core_axis_name="c", subcore_axis_name="s"),
        compiler_params=pltpu.CompilerParams(use_tc_tiling_on_sc=False),
        scratch_types=[
            pltpu.VMEM((EPW,), jnp.int32),
            pltpu.VMEM((EPW,), jnp.int32),
            pltpu.VMEM((2, GC, H), jnp.bfloat16),
            pltpu.VMEM((2, GC, H), jnp.bfloat16),
            pltpu.VMEM((2, GC, 16), jnp.float32),
            pltpu.VMEM((2, GC, 16), jnp.float32),
        ] + [pltpu.SemaphoreType.DMA] * 16,
    )(_sc_gather_body)


def _edge_body(g1_ref, g2_ref, c1_ref, c2_ref, w2_ref, wc1_ref, wc2_ref,
               we1r_ref, be1_ref, be2_ref, bc1_ref, ef_ref, tr_ref):
    cd = c1_ref[...] - c2_ref[...]
    lane = lax.broadcasted_iota(jnp.int32, cd.shape, 1)
    radial = jnp.sum(cd * cd, axis=1, keepdims=True)
    m = _silu(g1_ref[...].astype(jnp.float32) + g2_ref[...].astype(jnp.float32)
              + radial * we1r_ref[...] + be1_ref[...])
    ef = _silu(jnp.dot(m, w2_ref[...], preferred_element_type=jnp.float32)
               + be2_ref[...])
    p = _silu(jnp.dot(ef, wc1_ref[...], preferred_element_type=jnp.float32)
              + bc1_ref[...])
    scale = jnp.sum(p * wc2_ref[...], axis=1, keepdims=True)
    tr = cd * scale
    tr_ref[...] = jnp.where(lane == 3, 1.0, tr)
    ef_ref[...] = ef


def _edge_mlp(G1, G2, C1, C2, We2T, Wc1T, wc2row, we1rr, be1r, be2r, bc1r):
    grid = (E2 // BE,)
    return pl.pallas_call(
        _edge_body,
        grid=grid,
        in_specs=[
            pl.BlockSpec((BE, H), lambda i: (i, 0)),
            pl.BlockSpec((BE, H), lambda i: (i, 0)),
            pl.BlockSpec((BE, 16), lambda i: (i, 0)),
            pl.BlockSpec((BE, 16), lambda i: (i, 0)),
            pl.BlockSpec((H, H), lambda i: (0, 0)),
            pl.BlockSpec((H, H), lambda i: (0, 0)),
            pl.BlockSpec((1, H), lambda i: (0, 0)),
            pl.BlockSpec((1, H), lambda i: (0, 0)),
            pl.BlockSpec((1, H), lambda i: (0, 0)),
            pl.BlockSpec((1, H), lambda i: (0, 0)),
            pl.BlockSpec((1, H), lambda i: (0, 0)),
        ],
        out_specs=[
            pl.BlockSpec((BE, H), lambda i: (i, 0)),
            pl.BlockSpec((BE, 16), lambda i: (i, 0)),
        ],
        out_shape=[
            jax.ShapeDtypeStruct((E2, H), jnp.float32),
            jax.ShapeDtypeStruct((E2, 16), jnp.float32),
        ],
    )(G1, G2, C1, C2, We2T, Wc1T, wc2row, we1rr, be1r, be2r, bc1r)


NPS = N // 16


def _sc_scatter_body(ef, tr, rowi, zh, zt, agg_o, trn_o,
                     idx_l, ef_v, tr_v, agg_sh, trn_sh):
    c = lax.axis_index("c")
    s = lax.axis_index("s")
    wid = c * 16 + s
    base = wid * EPW

    pltpu.sync_copy(rowi.at[pl.ds(base, EPW)], idx_l)
    pltpu.sync_copy(zh.at[c].at[pl.ds(s * NPS, NPS)],
                    agg_sh.at[pl.ds(s * NPS, NPS)])
    pltpu.sync_copy(zt.at[c].at[pl.ds(s * NPS, NPS)],
                    trn_sh.at[pl.ds(s * NPS, NPS)])
    plsc.subcore_barrier()

    def chunk(i, carry):
        off = i * GC
        idx = idx_l.at[pl.ds(off, GC)]
        pltpu.sync_copy(ef.at[pl.ds(base + off, GC)], ef_v)
        pltpu.sync_copy(tr.at[pl.ds(base + off, GC)], tr_v)
        pltpu.sync_copy(ef_v, agg_sh.at[idx], add=True)
        pltpu.sync_copy(tr_v, trn_sh.at[idx], add=True)
        return carry

    lax.fori_loop(0, NFULL, chunk, 0)

    off = NFULL * GC
    idx = idx_l.at[pl.ds(off, REM)]
    pltpu.sync_copy(ef.at[pl.ds(base + off, REM)], ef_v.at[pl.ds(0, REM)])
    pltpu.sync_copy(tr.at[pl.ds(base + off, REM)], tr_v.at[pl.ds(0, REM)])
    pltpu.sync_copy(ef_v.at[pl.ds(0, REM)], agg_sh.at[idx], add=True)
    pltpu.sync_copy(tr_v.at[pl.ds(0, REM)], trn_sh.at[idx], add=True)
    plsc.subcore_barrier()

    pltpu.sync_copy(agg_sh.at[pl.ds(s * NPS, NPS)],
                    agg_o.at[c].at[pl.ds(s * NPS, NPS)])
    pltpu.sync_copy(trn_sh.at[pl.ds(s * NPS, NPS)],
                    trn_o.at[c].at[pl.ds(s * NPS, NPS)])


@functools.cache
def _sc_scatter():
    return functools.partial(
        pl.kernel,
        out_type=[jax.ShapeDtypeStruct((2, N, H), jnp.float32),
                  jax.ShapeDtypeStruct((2, N, 16), jnp.float32)],
        mesh=plsc.VectorSubcoreMesh(core_axis_name="c", subcore_axis_name="s"),
        compiler_params=pltpu.CompilerParams(use_tc_tiling_on_sc=False),
        scratch_types=[
            pltpu.VMEM((EPW,), jnp.int32),
            pltpu.VMEM((GC, H), jnp.float32),
            pltpu.VMEM((GC, 16), jnp.float32),
            pltpu.VMEM_SHARED((N, H), jnp.float32),
            pltpu.VMEM_SHARED((N, 16), jnp.float32),
        ],
    )(_sc_scatter_body)


def _node_body(h_ref, agg_ref, tr_ref, cp_ref, wn1a_ref, wn1b_ref, wn2_ref,
               bn1_ref, bn2_ref, ho_ref, co_ref):
    h = h_ref[...]
    agg = jnp.sum(agg_ref[...], axis=0)
    o = _silu(jnp.dot(h, wn1a_ref[...], preferred_element_type=jnp.float32)
              + jnp.dot(agg, wn1b_ref[...], preferred_element_type=jnp.float32)
              + bn1_ref[...])
    ho_ref[...] = h + jnp.dot(o, wn2_ref[...], preferred_element_type=jnp.float32) \
        + bn2_ref[...]
    tr = jnp.sum(tr_ref[...], axis=0)
    cnt = jnp.clip(tr[:, 3:4], 1.0, None)
    co_ref[...] = cp_ref[...] + tr / cnt


def _node_mlp(h, agg, tr, coordp, Wn1aT, Wn1bT, Wn2T, bn1r, bn2r):
    grid = (N // BN,)
    return pl.pallas_call(
        _node_body,
        grid=grid,
        in_specs=[
            pl.BlockSpec((BN, D), lambda i: (i, 0)),
            pl.BlockSpec((2, BN, H), lambda i: (0, i, 0)),
            pl.BlockSpec((2, BN, 16), lambda i: (0, i, 0)),
            pl.BlockSpec((BN, 16), lambda i: (i, 0)),
            pl.BlockSpec((D, H), lambda i: (0, 0)),
            pl.BlockSpec((H, H), lambda i: (0, 0)),
            pl.BlockSpec((H, D), lambda i: (0, 0)),
            pl.BlockSpec((1, H), lambda i: (0, 0)),
            pl.BlockSpec((1, D), lambda i: (0, 0)),
        ],
        out_specs=[
            pl.BlockSpec((BN, D), lambda i: (i, 0)),
            pl.BlockSpec((BN, 16), lambda i: (i, 0)),
        ],
        out_shape=[
            jax.ShapeDtypeStruct((N, D), jnp.float32),
            jax.ShapeDtypeStruct((N, 16), jnp.float32),
        ],
    )(h, agg, tr, coordp, Wn1aT, Wn1bT, Wn2T, bn1r, bn2r)


def kernel(h, coord, edge_index, We1, be1, We2, be2, Wn1, bn1, Wn2, bn2,
           Wc1, bc1, Wc2):
    row, col = edge_index[0], edge_index[1]
    WaT = We1[:, :D].T
    WbT = We1[:, D:2 * D].T
    we1r = We1[:, 2 * D]

    A, B = _precompute_ab(h, WaT, WbT)

    coordp = jnp.pad(coord, ((0, 0), (0, 13)))

    aggp = jnp.zeros((2, N, H), jnp.float32)
    trnp = jnp.zeros((2, N, 16), jnp.float32)
    for k in range(CH):
        rk = lax.slice_in_dim(row, k * E2, (k + 1) * E2)
        ck = lax.slice_in_dim(col, k * E2, (k + 1) * E2)
        G1, G2, C1, C2 = _sc_gather()(A, B, coordp, rk, ck)
        ef, tr = _edge_mlp(G1, G2, C1, C2, We2.T, Wc1.T, Wc2.reshape(1, H),
                           we1r.reshape(1, H), be1.reshape(1, H),
                           be2.reshape(1, H), bc1.reshape(1, H))
        aggp, trnp = _sc_scatter()(ef, tr, rk, aggp, trnp)

    h_out, co16 = _node_mlp(h, aggp, trnp, coordp, Wn1[:, :D].T, Wn1[:, D:].T,
                            Wn2.T, bn1.reshape(1, H), bn2.reshape(1, D))
    return (h_out, co16[:, :3])


# speedup vs baseline: 1.5122x; 1.5122x over previous
import functools

import jax
import jax.numpy as jnp
from jax import lax
from jax.experimental import pallas as pl
from jax.experimental.pallas import tpu as pltpu
from jax.experimental.pallas import tpu_sc as plsc

N = 10000
E = 320000
D = 128
H = 128

BN = 2000
BE = 2000

CH = 2
E2 = E // CH
NW = 32
EPW = E2 // NW
GC = 128
NFULL = EPW // GC
REM = EPW - NFULL * GC


def _silu(x):
    return x * jax.nn.sigmoid(x)


def _pre_body(h_ref, wa_ref, wb_ref, a_ref, b_ref):
    h = h_ref[...]
    a_ref[...] = jnp.dot(h, wa_ref[...], preferred_element_type=jnp.float32)
    b_ref[...] = jnp.dot(h, wb_ref[...], preferred_element_type=jnp.float32)


def _precompute_ab(h, WaT, WbT):
    grid = (N // BN,)
    return pl.pallas_call(
        _pre_body,
        grid=grid,
        in_specs=[
            pl.BlockSpec((BN, D), lambda i: (i, 0)),
            pl.BlockSpec((D, H), lambda i: (0, 0)),
            pl.BlockSpec((D, H), lambda i: (0, 0)),
        ],
        out_specs=[
            pl.BlockSpec((BN, H), lambda i: (i, 0)),
            pl.BlockSpec((BN, H), lambda i: (i, 0)),
        ],
        out_shape=[
            jax.ShapeDtypeStruct((N, H), jnp.float32),
            jax.ShapeDtypeStruct((N, H), jnp.float32),
        ],
    )(h, WaT, WbT)


def _sc_gather_body(a_t, b_t, c_t, rowi, coli, g1, g2, c1, c2,
                    idx1_l, idx2_l, g1_v, g2_v, c1_v, c2_v,
                    sem1, sem2, sem3, sem4, sem5, sem6, sem7, sem8,
                    sem9, sem10, sem11, sem12, sem13, sem14, sem15, sem16):
    wid = lax.axis_index("c") * 16 + lax.axis_index("s")
    base = wid * EPW

    pltpu.sync_copy(rowi.at[pl.ds(base, EPW)], idx1_l)
    pltpu.sync_copy(coli.at[pl.ds(base, EPW)], idx2_l)

    def gather_issue(i, slot, sems, n):
        off = i * GC
        i1 = idx1_l.at[pl.ds(off, n)]
        i2 = idx2_l.at[pl.ds(off, n)]
        return (
            pltpu.async_copy(a_t.at[i1], g1_v.at[slot].at[pl.ds(0, n)], sems[0]),
            pltpu.async_copy(b_t.at[i2], g2_v.at[slot].at[pl.ds(0, n)], sems[1]),
            pltpu.async_copy(c_t.at[i1], c1_v.at[slot].at[pl.ds(0, n)], sems[2]),
            pltpu.async_copy(c_t.at[i2], c2_v.at[slot].at[pl.ds(0, n)], sems[3]),
        )

    def wb_issue(i, slot, sems, n):
        off = base + i * GC
        return (
            pltpu.async_copy(g1_v.at[slot].at[pl.ds(0, n)], g1.at[pl.ds(off, n)], sems[0]),
            pltpu.async_copy(g2_v.at[slot].at[pl.ds(0, n)], g2.at[pl.ds(off, n)], sems[1]),
            pltpu.async_copy(c1_v.at[slot].at[pl.ds(0, n)], c1.at[pl.ds(off, n)], sems[2]),
            pltpu.async_copy(c2_v.at[slot].at[pl.ds(0, n)], c2.at[pl.ds(off, n)], sems[3]),
        )

    ga_sems = (sem1, sem2, sem3, sem4)
    gb_sems = (sem5, sem6, sem7, sem8)
    wa_sems = (sem9, sem10, sem11, sem12)
    wb_sems = (sem13, sem14, sem15, sem16)

    def pair(j, carry):
        i0 = 2 * j
        i1 = i0 + 1
        ga = gather_issue(i0, 0, ga_sems, GC)
        gb = gather_issue(i1, 1, gb_sems, GC)
        for cp in ga:
            cp.wait()
        wa = wb_issue(i0, 0, wa_sems, GC)
        for cp in gb:
            cp.wait()
        wb = wb_issue(i1, 1, wb_sems, GC)
        for cp in wa:
            cp.wait()
        for cp in wb:
            cp.wait()
        return carry

    lax.fori_loop(0, NFULL // 2, pair, 0)

    for i, n in ([(NFULL - 1, GC)] if NFULL % 2 else []) + [(NFULL, REM)]:
        ga = gather_issue(i, 0, ga_sems, n)
        for cp in ga:
            cp.wait()
        wa = wb_issue(i, 0, wa_sems, n)
        for cp in wa:
            cp.wait()


@functools.cache
def _sc_gather():
    return functools.partial(
        pl.kernel,
        out_type=[jax.ShapeDtypeStruct((E2, H), jnp.float32),
                  jax.ShapeDtypeStruct((E2, H), jnp.float32),
                  jax.ShapeDtypeStruct((E2, 16), jnp.float32),
                  jax.ShapeDtypeStruct((E2, 16), jnp.float32)],
        mesh=plsc.VectorSubcoreMesh(core_axis_name="c", subcore_axis_name="s"),
        compiler_params=pltpu.CompilerParams(use_tc_tiling_on_sc=False),
        scratch_types=[
            pltpu.VMEM((EPW,), jnp.int32),
            pltpu.VMEM((EPW,), jnp.int32),
            pltpu.VMEM((2, GC, H), jnp.float32),
            pltpu.VMEM((2, GC, H), jnp.float32),
            pltpu.VMEM((2, GC, 16), jnp.float32),
            pltpu.VMEM((2, GC, 16), jnp.float32),
        ] + [pltpu.SemaphoreType.DMA] * 16,
    )(_sc_gather_body)


def _edge_body(g1_ref, g2_ref, c1_ref, c2_ref, w2_ref, wc1_ref, wc2_ref,
               we1r_ref, be1_ref, be2_ref, bc1_ref, ef_ref, tr_ref):
    cd = c1_ref[...] - c2_ref[...]
    lane = lax.broadcasted_iota(jnp.int32, cd.shape, 1)
    radial = jnp.sum(cd * cd, axis=1, keepdims=True)
    m = _silu(g1_ref[...] + g2_ref[...] + radial * we1r_ref[...] + be1_ref[...])
    ef = _silu(jnp.dot(m.astype(jnp.bfloat16),
                       w2_ref[...].astype(jnp.bfloat16),
                       preferred_element_type=jnp.float32)
               + be2_ref[...])
    p = _silu(jnp.dot(ef.astype(jnp.bfloat16),
                      wc1_ref[...].astype(jnp.bfloat16),
                      preferred_element_type=jnp.float32)
              + bc1_ref[...])
    scale = jnp.sum(p * wc2_ref[...], axis=1, keepdims=True)
    tr = cd * scale
    tr_ref[...] = jnp.where(lane == 3, 1.0, tr)
    ef_ref[...] = ef


def _edge_mlp(G1, G2, C1, C2, We2T, Wc1T, wc2row, we1rr, be1r, be2r, bc1r):
    grid = (E2 // BE,)
    return pl.pallas_call(
        _edge_body,
        grid=grid,
        in_specs=[
            pl.BlockSpec((BE, H), lambda i: (i, 0)),
            pl.BlockSpec((BE, H), lambda i: (i, 0)),
            pl.BlockSpec((BE, 16), lambda i: (i, 0)),
            pl.BlockSpec((BE, 16), lambda i: (i, 0)),
            pl.BlockSpec((H, H), lambda i: (0, 0)),
            pl.BlockSpec((H, H), lambda i: (0, 0)),
            pl.BlockSpec((1, H), lambda i: (0, 0)),
            pl.BlockSpec((1, H), lambda i: (0, 0)),
            pl.BlockSpec((1, H), lambda i: (0, 0)),
            pl.BlockSpec((1, H), lambda i: (0, 0)),
            pl.BlockSpec((1, H), lambda i: (0, 0)),
        ],
        out_specs=[
            pl.BlockSpec((BE, H), lambda i: (i, 0)),
            pl.BlockSpec((BE, 16), lambda i: (i, 0)),
        ],
        out_shape=[
            jax.ShapeDtypeStruct((E2, H), jnp.float32),
            jax.ShapeDtypeStruct((E2, 16), jnp.float32),
        ],
    )(G1, G2, C1, C2, We2T, Wc1T, wc2row, we1rr, be1r, be2r, bc1r)


NPS = N // 16


def _sc_scatter_body(ef, tr, rowi, zh, zt, agg_o, trn_o,
                     idx_l, ef_v, tr_v, agg_sh, trn_sh):
    c = lax.axis_index("c")
    s = lax.axis_index("s")
    wid = c * 16 + s
    base = wid * EPW

    pltpu.sync_copy(rowi.at[pl.ds(base, EPW)], idx_l)
    pltpu.sync_copy(zh.at[c].at[pl.ds(s * NPS, NPS)],
                    agg_sh.at[pl.ds(s * NPS, NPS)])
    pltpu.sync_copy(zt.at[c].at[pl.ds(s * NPS, NPS)],
                    trn_sh.at[pl.ds(s * NPS, NPS)])
    plsc.subcore_barrier()

    def chunk(i, carry):
        off = i * GC
        idx = idx_l.at[pl.ds(off, GC)]
        pltpu.sync_copy(ef.at[pl.ds(base + off, GC)], ef_v)
        pltpu.sync_copy(tr.at[pl.ds(base + off, GC)], tr_v)
        pltpu.sync_copy(ef_v, agg_sh.at[idx], add=True)
        pltpu.sync_copy(tr_v, trn_sh.at[idx], add=True)
        return carry

    lax.fori_loop(0, NFULL, chunk, 0)

    off = NFULL * GC
    idx = idx_l.at[pl.ds(off, REM)]
    pltpu.sync_copy(ef.at[pl.ds(base + off, REM)], ef_v.at[pl.ds(0, REM)])
    pltpu.sync_copy(tr.at[pl.ds(base + off, REM)], tr_v.at[pl.ds(0, REM)])
    pltpu.sync_copy(ef_v.at[pl.ds(0, REM)], agg_sh.at[idx], add=True)
    pltpu.sync_copy(tr_v.at[pl.ds(0, REM)], trn_sh.at[idx], add=True)
    plsc.subcore_barrier()

    pltpu.sync_copy(agg_sh.at[pl.ds(s * NPS, NPS)],
                    agg_o.at[c].at[pl.ds(s * NPS, NPS)])
    pltpu.sync_copy(trn_sh.at[pl.ds(s * NPS, NPS)],
                    trn_o.at[c].at[pl.ds(s * NPS, NPS)])


@functools.cache
def _sc_scatter():
    return functools.partial(
        pl.kernel,
        out_type=[jax.ShapeDtypeStruct((2, N, H), jnp.float32),
                  jax.ShapeDtypeStruct((2, N, 16), jnp.float32)],
        mesh=plsc.VectorSubcoreMesh(core_axis_name="c", subcore_axis_name="s"),
        compiler_params=pltpu.CompilerParams(use_tc_tiling_on_sc=False),
        scratch_types=[
            pltpu.VMEM((EPW,), jnp.int32),
            pltpu.VMEM((GC, H), jnp.float32),
            pltpu.VMEM((GC, 16), jnp.float32),
            pltpu.VMEM_SHARED((N, H), jnp.float32),
            pltpu.VMEM_SHARED((N, 16), jnp.float32),
        ],
    )(_sc_scatter_body)


def _node_body(h_ref, agg_ref, tr_ref, cp_ref, wn1a_ref, wn1b_ref, wn2_ref,
               bn1_ref, bn2_ref, ho_ref, co_ref):
    h = h_ref[...]
    agg = jnp.sum(agg_ref[...], axis=0)
    o = _silu(jnp.dot(h, wn1a_ref[...], preferred_element_type=jnp.float32)
              + jnp.dot(agg, wn1b_ref[...], preferred_element_type=jnp.float32)
              + bn1_ref[...])
    ho_ref[...] = h + jnp.dot(o, wn2_ref[...], preferred_element_type=jnp.float32) \
        + bn2_ref[...]
    tr = jnp.sum(tr_ref[...], axis=0)
    cnt = jnp.clip(tr[:, 3:4], 1.0, None)
    co_ref[...] = cp_ref[...] + tr / cnt


def _node_mlp(h, agg, tr, coordp, Wn1aT, Wn1bT, Wn2T, bn1r, bn2r):
    grid = (N // BN,)
    return pl.pallas_call(
        _node_body,
        grid=grid,
        in_specs=[
            pl.BlockSpec((BN, D), lambda i: (i, 0)),
            pl.BlockSpec((2, BN, H), lambda i: (0, i, 0)),
            pl.BlockSpec((2, BN, 16), lambda i: (0, i, 0)),
            pl.BlockSpec((BN, 16), lambda i: (i, 0)),
            pl.BlockSpec((D, H), lambda i: (0, 0)),
            pl.BlockSpec((H, H), lambda i: (0, 0)),
            pl.BlockSpec((H, D), lambda i: (0, 0)),
            pl.BlockSpec((1, H), lambda i: (0, 0)),
            pl.BlockSpec((1, D), lambda i: (0, 0)),
        ],
        out_specs=[
            pl.BlockSpec((BN, D), lambda i: (i, 0)),
            pl.BlockSpec((BN, 16), lambda i: (i, 0)),
        ],
        out_shape=[
            jax.ShapeDtypeStruct((N, D), jnp.float32),
            jax.ShapeDtypeStruct((N, 16), jnp.float32),
        ],
    )(h, agg, tr, coordp, Wn1aT, Wn1bT, Wn2T, bn1r, bn2r)


def kernel(h, coord, edge_index, We1, be1, We2, be2, Wn1, bn1, Wn2, bn2,
           Wc1, bc1, Wc2):
    row, col = edge_index[0], edge_index[1]
    WaT = We1[:, :D].T
    WbT = We1[:, D:2 * D].T
    we1r = We1[:, 2 * D]

    A, B = _precompute_ab(h, WaT, WbT)

    coordp = jnp.pad(coord, ((0, 0), (0, 13)))

    aggp = jnp.zeros((2, N, H), jnp.float32)
    trnp = jnp.zeros((2, N, 16), jnp.float32)
    for k in range(CH):
        rk = lax.slice_in_dim(row, k * E2, (k + 1) * E2)
        ck = lax.slice_in_dim(col, k * E2, (k + 1) * E2)
        G1, G2, C1, C2 = _sc_gather()(A, B, coordp, rk, ck)
        ef, tr = _edge_mlp(G1, G2, C1, C2, We2.T, Wc1.T, Wc2.reshape(1, H),
                           we1r.reshape(1, H), be1.reshape(1, H),
                           be2.reshape(1, H), bc1.reshape(1, H))
        aggp, trnp = _sc_scatter()(ef, tr, rk, aggp, trnp)

    h_out, co16 = _node_mlp(h, aggp, trnp, coordp, Wn1[:, :D].T, Wn1[:, D:].T,
                            Wn2.T, bn1.reshape(1, H), bn2.reshape(1, D))
    return (h_out, co16[:, :3])


# speedup vs baseline: 1.5251x; 1.0085x over previous
import functools

import jax
import jax.numpy as jnp
from jax import lax
from jax.experimental import pallas as pl
from jax.experimental.pallas import tpu as pltpu
from jax.experimental.pallas import tpu_sc as plsc

N = 10000
E = 320000
D = 128
H = 128

BN = 2000
BE = 2000

CH = 2
E2 = E // CH
NW = 32
EPW = E2 // NW
GC = 128
NFULL = EPW // GC
REM = EPW - NFULL * GC


def _silu(x):
    return x * jax.nn.sigmoid(x)


def _pre_body(h_ref, wa_ref, wb_ref, a_ref, b_ref):
    h = h_ref[...]
    a_ref[...] = jnp.dot(h, wa_ref[...], preferred_element_type=jnp.float32)
    b_ref[...] = jnp.dot(h, wb_ref[...], preferred_element_type=jnp.float32)


def _precompute_ab(h, WaT, WbT):
    grid = (N // BN,)
    return pl.pallas_call(
        _pre_body,
        grid=grid,
        in_specs=[
            pl.BlockSpec((BN, D), lambda i: (i, 0)),
            pl.BlockSpec((D, H), lambda i: (0, 0)),
            pl.BlockSpec((D, H), lambda i: (0, 0)),
        ],
        out_specs=[
            pl.BlockSpec((BN, H), lambda i: (i, 0)),
            pl.BlockSpec((BN, H), lambda i: (i, 0)),
        ],
        out_shape=[
            jax.ShapeDtypeStruct((N, H), jnp.float32),
            jax.ShapeDtypeStruct((N, H), jnp.float32),
        ],
    )(h, WaT, WbT)


def _sc_gather_body(a_t, b_t, c_t, rowi, coli, g1, g2, c1, c2,
                    idx1_l, idx2_l, g1_v, g2_v, c1_v, c2_v,
                    sem1, sem2, sem3, sem4, sem5, sem6, sem7, sem8,
                    sem9, sem10, sem11, sem12, sem13, sem14, sem15, sem16):
    wid = lax.axis_index("c") * 16 + lax.axis_index("s")
    base = wid * EPW

    pltpu.sync_copy(rowi.at[pl.ds(base, EPW)], idx1_l)
    pltpu.sync_copy(coli.at[pl.ds(base, EPW)], idx2_l)

    def gather_issue(i, slot, sems, n):
        off = i * GC
        i1 = idx1_l.at[pl.ds(off, n)]
        i2 = idx2_l.at[pl.ds(off, n)]
        return (
            pltpu.async_copy(a_t.at[i1], g1_v.at[slot].at[pl.ds(0, n)], sems[0]),
            pltpu.async_copy(b_t.at[i2], g2_v.at[slot].at[pl.ds(0, n)], sems[1]),
            pltpu.async_copy(c_t.at[i1], c1_v.at[slot].at[pl.ds(0, n)], sems[2]),
            pltpu.async_copy(c_t.at[i2], c2_v.at[slot].at[pl.ds(0, n)], sems[3]),
        )

    def wb_issue(i, slot, sems, n):
        off = base + i * GC
        return (
            pltpu.async_copy(g1_v.at[slot].at[pl.ds(0, n)], g1.at[pl.ds(off, n)], sems[0]),
            pltpu.async_copy(g2_v.at[slot].at[pl.ds(0, n)], g2.at[pl.ds(off, n)], sems[1]),
            pltpu.async_copy(c1_v.at[slot].at[pl.ds(0, n)], c1.at[pl.ds(off, n)], sems[2]),
            pltpu.async_copy(c2_v.at[slot].at[pl.ds(0, n)], c2.at[pl.ds(off, n)], sems[3]),
        )

    ga_sems = (sem1, sem2, sem3, sem4)
    gb_sems = (sem5, sem6, sem7, sem8)
    wa_sems = (sem9, sem10, sem11, sem12)
    wb_sems = (sem13, sem14, sem15, sem16)

    def pair(j, carry):
        i0 = 2 * j
        i1 = i0 + 1
        ga = gather_issue(i0, 0, ga_sems, GC)
        gb = gather_issue(i1, 1, gb_sems, GC)
        for cp in ga:
            cp.wait()
        wa = wb_issue(i0, 0, wa_sems, GC)
        for cp in gb:
            cp.wait()
        wb = wb_issue(i1, 1, wb_sems, GC)
        for cp in wa:
            cp.wait()
        for cp in wb:
            cp.wait()
        return carry

    lax.fori_loop(0, NFULL // 2, pair, 0)

    for i, n in ([(NFULL - 1, GC)] if NFULL % 2 else []) + [(NFULL, REM)]:
        ga = gather_issue(i, 0, ga_sems, n)
        for cp in ga:
            cp.wait()
        wa = wb_issue(i, 0, wa_sems, n)
        for cp in wa:
            cp.wait()


@functools.cache
def _sc_gather():
    return functools.partial(
        pl.kernel,
        out_type=[jax.ShapeDtypeStruct((E2, H), jnp.float32),
                  jax.ShapeDtypeStruct((E2, H), jnp.float32),
                  jax.ShapeDtypeStruct((E2, 16), jnp.float32),
                  jax.ShapeDtypeStruct((E2, 16), jnp.float32)],
        mesh=plsc.VectorSubcoreMesh(core_axis_name="c", subcore_axis_name="s"),
        compiler_params=pltpu.CompilerParams(use_tc_tiling_on_sc=False),
        scratch_types=[
            pltpu.VMEM((EPW,), jnp.int32),
            pltpu.VMEM((EPW,), jnp.int32),
            pltpu.VMEM((2, GC, H), jnp.float32),
            pltpu.VMEM((2, GC, H), jnp.float32),
            pltpu.VMEM((2, GC, 16), jnp.float32),
            pltpu.VMEM((2, GC, 16), jnp.float32),
        ] + [pltpu.SemaphoreType.DMA] * 16,
    )(_sc_gather_body)


def _edge_body(g1_ref, g2_ref, c1_ref, c2_ref, w2_ref, wc1_ref, wc2_ref,
               we1r_ref, be1_ref, be2_ref, bc1_ref, ef_ref, tr_ref):
    cd = c1_ref[...] - c2_ref[...]
    lane = lax.broadcasted_iota(jnp.int32, cd.shape, 1)
    radial = jnp.sum(cd * cd, axis=1, keepdims=True)
    m = _silu(g1_ref[...] + g2_ref[...] + radial * we1r_ref[...] + be1_ref[...])
    ef = _silu(jnp.dot(m, w2_ref[...], preferred_element_type=jnp.float32)
               + be2_ref[...])
    p = _silu(jnp.dot(ef, wc1_ref[...], preferred_element_type=jnp.float32)
              + bc1_ref[...])
    scale = jnp.sum(p * wc2_ref[...], axis=1, keepdims=True)
    tr = cd * scale
    tr_ref[...] = jnp.where(lane == 3, 1.0, tr)
    ef_ref[...] = ef


def _edge_mlp(G1, G2, C1, C2, We2T, Wc1T, wc2row, we1rr, be1r, be2r, bc1r):
    grid = (E2 // BE,)
    return pl.pallas_call(
        _edge_body,
        grid=grid,
        in_specs=[
            pl.BlockSpec((BE, H), lambda i: (i, 0)),
            pl.BlockSpec((BE, H), lambda i: (i, 0)),
            pl.BlockSpec((BE, 16), lambda i: (i, 0)),
            pl.BlockSpec((BE, 16), lambda i: (i, 0)),
            pl.BlockSpec((H, H), lambda i: (0, 0)),
            pl.BlockSpec((H, H), lambda i: (0, 0)),
            pl.BlockSpec((1, H), lambda i: (0, 0)),
            pl.BlockSpec((1, H), lambda i: (0, 0)),
            pl.BlockSpec((1, H), lambda i: (0, 0)),
            pl.BlockSpec((1, H), lambda i: (0, 0)),
            pl.BlockSpec((1, H), lambda i: (0, 0)),
        ],
        out_specs=[
            pl.BlockSpec((BE, H), lambda i: (i, 0)),
            pl.BlockSpec((BE, 16), lambda i: (i, 0)),
        ],
        out_shape=[
            jax.ShapeDtypeStruct((E2, H), jnp.float32),
            jax.ShapeDtypeStruct((E2, 16), jnp.float32),
        ],
    )(G1, G2, C1, C2, We2T, Wc1T, wc2row, we1rr, be1r, be2r, bc1r)


NPS = N // 16


def _sc_scatter_body(ef, tr, rowi, zh, zt, agg_o, trn_o,
                     idx_l, ef_v, tr_v, agg_sh, trn_sh):
    c = lax.axis_index("c")
    s = lax.axis_index("s")
    wid = c * 16 + s
    base = wid * EPW

    pltpu.sync_copy(rowi.at[pl.ds(base, EPW)], idx_l)
    pltpu.sync_copy(zh.at[c].at[pl.ds(s * NPS, NPS)],
                    agg_sh.at[pl.ds(s * NPS, NPS)])
    pltpu.sync_copy(zt.at[c].at[pl.ds(s * NPS, NPS)],
                    trn_sh.at[pl.ds(s * NPS, NPS)])
    plsc.subcore_barrier()

    def chunk(i, carry):
        off = i * GC
        idx = idx_l.at[pl.ds(off, GC)]
        pltpu.sync_copy(ef.at[pl.ds(base + off, GC)], ef_v)
        pltpu.sync_copy(tr.at[pl.ds(base + off, GC)], tr_v)
        pltpu.sync_copy(ef_v, agg_sh.at[idx], add=True)
        pltpu.sync_copy(tr_v, trn_sh.at[idx], add=True)
        return carry

    lax.fori_loop(0, NFULL, chunk, 0)

    off = NFULL * GC
    idx = idx_l.at[pl.ds(off, REM)]
    pltpu.sync_copy(ef.at[pl.ds(base + off, REM)], ef_v.at[pl.ds(0, REM)])
    pltpu.sync_copy(tr.at[pl.ds(base + off, REM)], tr_v.at[pl.ds(0, REM)])
    pltpu.sync_copy(ef_v.at[pl.ds(0, REM)], agg_sh.at[idx], add=True)
    pltpu.sync_copy(tr_v.at[pl.ds(0, REM)], trn_sh.at[idx], add=True)
    plsc.subcore_barrier()

    pltpu.sync_copy(agg_sh.at[pl.ds(s * NPS, NPS)],
                    agg_o.at[c].at[pl.ds(s * NPS, NPS)])
    pltpu.sync_copy(trn_sh.at[pl.ds(s * NPS, NPS)],
                    trn_o.at[c].at[pl.ds(s * NPS, NPS)])


@functools.cache
def _sc_scatter():
    return functools.partial(
        pl.kernel,
        out_type=[jax.ShapeDtypeStruct((2, N, H), jnp.float32),
                  jax.ShapeDtypeStruct((2, N, 16), jnp.float32)],
        mesh=plsc.VectorSubcoreMesh(core_axis_name="c", subcore_axis_name="s"),
        compiler_params=pltpu.CompilerParams(use_tc_tiling_on_sc=False),
        scratch_types=[
            pltpu.VMEM((EPW,), jnp.int32),
            pltpu.VMEM((GC, H), jnp.float32),
            pltpu.VMEM((GC, 16), jnp.float32),
            pltpu.VMEM_SHARED((N, H), jnp.float32),
            pltpu.VMEM_SHARED((N, 16), jnp.float32),
        ],
    )(_sc_scatter_body)


def _node_body(h_ref, agg_ref, tr_ref, cp_ref, wn1a_ref, wn1b_ref, wn2_ref,
               bn1_ref, bn2_ref, ho_ref, co_ref):
    h = h_ref[...]
    agg = jnp.sum(agg_ref[...], axis=0)
    o = _silu(jnp.dot(h, wn1a_ref[...], preferred_element_type=jnp.float32)
              + jnp.dot(agg, wn1b_ref[...], preferred_element_type=jnp.float32)
              + bn1_ref[...])
    ho_ref[...] = h + jnp.dot(o, wn2_ref[...], preferred_element_type=jnp.float32) \
        + bn2_ref[...]
    tr = jnp.sum(tr_ref[...], axis=0)
    cnt = jnp.clip(tr[:, 3:4], 1.0, None)
    co_ref[...] = cp_ref[...] + tr / cnt


def _node_mlp(h, agg, tr, coordp, Wn1aT, Wn1bT, Wn2T, bn1r, bn2r):
    grid = (N // BN,)
    return pl.pallas_call(
        _node_body,
        grid=grid,
        in_specs=[
            pl.BlockSpec((BN, D), lambda i: (i, 0)),
            pl.BlockSpec((2, BN, H), lambda i: (0, i, 0)),
            pl.BlockSpec((2, BN, 16), lambda i: (0, i, 0)),
            pl.BlockSpec((BN, 16), lambda i: (i, 0)),
            pl.BlockSpec((D, H), lambda i: (0, 0)),
            pl.BlockSpec((H, H), lambda i: (0, 0)),
            pl.BlockSpec((H, D), lambda i: (0, 0)),
            pl.BlockSpec((1, H), lambda i: (0, 0)),
            pl.BlockSpec((1, D), lambda i: (0, 0)),
        ],
        out_specs=[
            pl.BlockSpec((BN, D), lambda i: (i, 0)),
            pl.BlockSpec((BN, 16), lambda i: (i, 0)),
        ],
        out_shape=[
            jax.ShapeDtypeStruct((N, D), jnp.float32),
            jax.ShapeDtypeStruct((N, 16), jnp.float32),
        ],
    )(h, agg, tr, coordp, Wn1aT, Wn1bT, Wn2T, bn1r, bn2r)


def kernel(h, coord, edge_index, We1, be1, We2, be2, Wn1, bn1, Wn2, bn2,
           Wc1, bc1, Wc2):
    row, col = edge_index[0], edge_index[1]
    WaT = We1[:, :D].T
    WbT = We1[:, D:2 * D].T
    we1r = We1[:, 2 * D]

    A, B = _precompute_ab(h, WaT, WbT)

    coordp = jnp.pad(coord, ((0, 0), (0, 13)))

    aggp = jnp.zeros((2, N, H), jnp.float32)
    trnp = jnp.zeros((2, N, 16), jnp.float32)
    for k in range(CH):
        rk = lax.slice_in_dim(row, k * E2, (k + 1) * E2)
        ck = lax.slice_in_dim(col, k * E2, (k + 1) * E2)
        G1, G2, C1, C2 = _sc_gather()(A, B, coordp, rk, ck)
        ef, tr = _edge_mlp(G1, G2, C1, C2, We2.T, Wc1.T, Wc2.reshape(1, H),
                           we1r.reshape(1, H), be1.reshape(1, H),
                           be2.reshape(1, H), bc1.reshape(1, H))
        aggp, trnp = _sc_scatter()(ef, tr, rk, aggp, trnp)

    h_out, co16 = _node_mlp(h, aggp, trnp, coordp, Wn1[:, :D].T, Wn1[:, D:].T,
                            Wn2.T, bn1.reshape(1, H), bn2.reshape(1, D))
    return (h_out, co16[:, :3])
